# baseline (device time: 15121 ns/iter reference)
import jax
import jax.numpy as jnp
from jax import lax
from jax.experimental import pallas as pl
from jax.experimental.pallas import tpu as pltpu

N_DEV = 8
N_STEPS = 3
BLK_ROWS = 8


def kernel(x):
    m, n = x.shape
    nblk = m // BLK_ROWS

    def body(x_ref, out_ref, incl_ref, excl_ref, recv_ref, send_sems, recv_sems):
        my = lax.axis_index("i")
        f32 = jnp.float32

        xv = x_ref[...].reshape(nblk, BLK_ROWS, n)

        t = xv[:, :4, :] * xv[:, 4:, :]
        t = t[:, :2, :] * t[:, 2:, :]
        p8 = t[:, 0, :] * t[:, 1, :]

        b = p8
        d = 1
        while d < nblk:
            b = b * jnp.concatenate(
                [jnp.ones((d, n), f32), b[: nblk - d, :]], axis=0
            )
            d *= 2
        incl_ref[0:1, :] = b[nblk - 1 : nblk, :]
        excl_ref[0, :] = jnp.ones((n,), f32)

        for s in range(N_STEPS):
            dd = 1 << s
            sends = my + dd < N_DEV
            recvs = my - dd >= 0
            copy = pltpu.make_async_remote_copy(
                src_ref=incl_ref,
                dst_ref=recv_ref.at[s],
                send_sem=send_sems.at[s],
                recv_sem=recv_sems.at[s],
                device_id=(jnp.minimum(my + dd, N_DEV - 1),),
                device_id_type=pl.DeviceIdType.MESH,
            )

            @pl.when(sends)
            def _():
                copy.start()

            xv = xv * jnp.concatenate(
                [
                    jnp.ones((nblk, dd, n), f32),
                    xv[:, : BLK_ROWS - dd, :],
                ],
                axis=1,
            )

            @pl.when(recvs)
            def _():
                copy.wait_recv()

            @pl.when(sends)
            def _():
                copy.wait_send()

            @pl.when(recvs)
            def _():
                r = recv_ref[s, 0, :]
                incl_ref[0, :] = incl_ref[0, :] * r
                excl_ref[0, :] = excl_ref[0, :] * r

        bex = (
            jnp.concatenate(
                [jnp.ones((1, n), f32), b[: nblk - 1, :]], axis=0
            )
            * excl_ref[0:1, :]
        )
        out_ref[...] = (xv * bex[:, None, :]).reshape(m, n)

    return pl.pallas_call(
        body,
        out_shape=jax.ShapeDtypeStruct((m, n), jnp.float32),
        in_specs=[pl.BlockSpec(memory_space=pltpu.VMEM)],
        out_specs=pl.BlockSpec(memory_space=pltpu.VMEM),
        scratch_shapes=[
            pltpu.VMEM((1, n), jnp.float32),
            pltpu.VMEM((1, n), jnp.float32),
            pltpu.VMEM((N_STEPS, 1, n), jnp.float32),
            pltpu.SemaphoreType.DMA((N_STEPS,)),
            pltpu.SemaphoreType.DMA((N_STEPS,)),
        ],
    )(x)
